# baseline (speedup 1.0000x reference)
import jax
import jax.numpy as jnp
from jax import lax
from jax.experimental import pallas as pl
from jax.experimental.pallas import tpu as pltpu

NUM_TOPICS = 512
TOP_K = 8
DIM = 1024


def _body(tvT_ref, out_hbm, zbuf, sheet, sems):
    batch = out_hbm.shape[0]
    max_len = out_hbm.shape[1]

    zbuf[...] = jnp.zeros(zbuf.shape, jnp.float32)
    half = (max_len - DIM) // 2
    zcopies = []
    for b in range(batch):
        for h in range(2):
            c = pltpu.make_async_copy(
                zbuf.at[pl.ds(h * half, half), :],
                out_hbm.at[b, pl.ds(DIM + h * half, half), :],
                sems.at[2 * b + h],
            )
            c.start()
            zcopies.append(c)

    x0 = tvT_ref[...]
    iota = lax.broadcasted_iota(jnp.int32, x0.shape, 0)
    neg_inf = jnp.float32(-jnp.inf)
    x = x0
    for _ in range(TOP_K):
        m = jnp.max(x, axis=0, keepdims=True)
        cand = jnp.where(x == m, iota, jnp.int32(DIM))
        amin = jnp.min(cand, axis=0, keepdims=True)
        x = jnp.where(iota == amin, neg_inf, x)
    sheet[...] = jnp.where(x != x0, jnp.float32(1.0), jnp.float32(0.0))

    scopies = []
    for b in range(batch):
        for h in range(2):
            c = pltpu.make_async_copy(
                sheet.at[pl.ds(h * (DIM // 2), DIM // 2), :],
                out_hbm.at[b, pl.ds(h * (DIM // 2), DIM // 2), :],
                sems.at[2 * batch + 2 * b + h],
            )
            c.start()
            scopies.append(c)

    for c in zcopies + scopies:
        c.wait()


def kernel(inputs, topic_vectors):
    _, batch, max_len, _ = inputs.shape
    tvT = topic_vectors.T
    out = pl.pallas_call(
        _body,
        in_specs=[pl.BlockSpec(memory_space=pltpu.MemorySpace.VMEM)],
        out_specs=pl.BlockSpec(memory_space=pltpu.MemorySpace.HBM),
        out_shape=jax.ShapeDtypeStruct((batch, max_len, NUM_TOPICS), jnp.float32),
        scratch_shapes=[
            pltpu.VMEM((max_len - DIM, NUM_TOPICS), jnp.float32),
            pltpu.VMEM((DIM, NUM_TOPICS), jnp.float32),
            pltpu.SemaphoreType.DMA((4 * batch,)),
        ],
    )(tvT)
    return out
